# Initial kernel scaffold; baseline (speedup 1.0000x reference)
import jax
import jax.numpy as jnp
from jax.experimental import pallas as pl


def kernel(inputs, targets):
    raise NotImplementedError("write your pallas kernel here")


# speedup vs baseline: 1.2844x; 1.2844x over previous
import functools

import jax
import jax.numpy as jnp
from jax import lax
from jax.experimental import pallas as pl
from jax.experimental.pallas import tpu as pltpu


def _round_up(x, m):
    return ((x + m - 1) // m) * m


def _hard_mine_kernel(xall_ref, sq_row_ref, sq_col_ref, t_col_ref, t_row_ref,
                      hinge_ref, corr_ref, *, n, bn, margin):
    i = pl.program_id(0)
    xb = xall_ref[pl.ds(i * bn, bn), :]
    xall = xall_ref[...]

    gram = lax.dot_general(xb, xall, (((1,), (1,)), ((), ())),
                           preferred_element_type=jnp.float32)

    part = sq_col_ref[...] - 2.0 * gram
    same_id = t_col_ref[...] == t_row_ref[...]
    ap = jnp.max(jnp.where(same_id, part, -jnp.inf), axis=1, keepdims=True)
    an = jnp.min(jnp.where(same_id, jnp.inf, part), axis=1, keepdims=True)

    sq_row = sq_row_ref[...]
    dist_ap = jnp.sqrt(jnp.maximum(ap + sq_row, 1e-12))
    dist_an = jnp.sqrt(jnp.maximum(an + sq_row, 1e-12))

    hinge = jnp.maximum(dist_ap - dist_an + margin, 0.0)
    corr = (dist_an >= dist_ap).astype(jnp.int32)

    if n % bn:
        row_ids = i * bn + lax.broadcasted_iota(jnp.int32, (bn, 1), 0)
        row_valid = row_ids < n
        hinge = jnp.where(row_valid, hinge, 0.0)
        corr = jnp.where(row_valid, corr, 0)

    hinge_ref[...] = hinge
    corr_ref[...] = corr


def _triplet_call(n, n_pad, d_pad, bn, margin):
    body = functools.partial(_hard_mine_kernel, n=n, bn=bn,
                             margin=float(margin))

    def resident(shape):
        return pl.BlockSpec(shape, lambda i: tuple(0 for _ in shape),
                            pipeline_mode=pl.Buffered(1))

    return pl.pallas_call(
        body,
        grid=(n_pad // bn,),
        in_specs=[
            resident((n_pad, d_pad)),
            pl.BlockSpec((bn, 1), lambda i: (i, 0)),
            resident((1, n_pad)),
            pl.BlockSpec((bn, 1), lambda i: (i, 0)),
            resident((1, n_pad)),
        ],
        out_specs=[
            pl.BlockSpec((bn, 1), lambda i: (i, 0)),
            pl.BlockSpec((bn, 1), lambda i: (i, 0)),
        ],
        out_shape=[
            jax.ShapeDtypeStruct((n_pad, 1), jnp.float32),
            jax.ShapeDtypeStruct((n_pad, 1), jnp.int32),
        ],
        compiler_params=pltpu.CompilerParams(
            dimension_semantics=("parallel",),
            vmem_limit_bytes=64 * 1024 * 1024),
    )


def kernel(inputs, targets):
    margin = 0.3
    x = jnp.asarray(inputs, jnp.float32)
    t = jnp.asarray(targets, jnp.int32)
    n, d = x.shape

    bn = 256 if n >= 256 else _round_up(min(n, 128), 8)
    n_pad = _round_up(n, bn)
    d_pad = _round_up(d, 128)

    if (n_pad, d_pad) != (n, d):
        xp = jnp.zeros((n_pad, d_pad), jnp.float32).at[:n, :d].set(x)
    else:
        xp = x

    sq = jnp.sum(xp * xp, axis=1)
    if n_pad != n:
        col_valid = jnp.arange(n_pad) < n
        sq_col = jnp.where(col_valid, sq, jnp.float32(1e30)).reshape(1, n_pad)
        sentinel = jnp.min(t) - jnp.int32(1)
        tp = jnp.full((n_pad,), sentinel, jnp.int32).at[:n].set(t)
    else:
        sq_col = sq.reshape(1, n_pad)
        tp = t
    sq_row = sq.reshape(n_pad, 1)

    x16 = xp.astype(jnp.bfloat16)

    hinge, corr = _triplet_call(n, n_pad, d_pad, bn, margin)(
        x16, sq_row, sq_col, tp.reshape(n_pad, 1), tp.reshape(1, n_pad))

    loss = jnp.sum(hinge) / jnp.float32(n)
    correct = jnp.sum(corr)
    return loss, correct
